# baseline (speedup 1.0000x reference)
import functools

import jax
import jax.numpy as jnp
import numpy as np
from jax import lax
from jax.experimental import pallas as pl
from jax.experimental.pallas import tpu as pltpu
from jax.experimental.pallas import tpu_sc as plsc

N = 10000
E = 160000
EP = 163840
D = 256
ED = 16
H = 256
G = 64
L = 3

_CK = 64
_CH = 160
_ROWS_PER_TILE = N // 16
_TRASH = N
_SC_ROWS = N + 8


def _sc_aggr_body(h2f, ea2f, src2, dst2, out,
                  src8, adj8, dst8, h0, h1, m0, m1, tbuf,
                  aggr_sh, se0, se1, sg0, sg1, ssc):
    c = lax.axis_index("c")
    s = lax.axis_index("s")
    hb = (h0, h1)
    mb = (m0, m1)
    se = (se0, se1)
    sg = (sg0, sg1)

    def zrow(i, carry):
        for j in range(8):
            tbuf[i, pl.ds(j * 16, 16)] = jnp.zeros((16,), jnp.float32)
        return carry
    lax.fori_loop(0, _CK, zrow, 0)
    zbase = s * 624
    for k in range(10):
        off = k * 64
        nr = 64 if k < 9 else 48
        pltpu.sync_copy(tbuf.at[pl.ds(0, nr)],
                        aggr_sh.at[pl.ds(zbase + off, nr)])

    @pl.when(s == 0)
    def _():
        pltpu.sync_copy(tbuf.at[pl.ds(0, 24)], aggr_sh.at[pl.ds(9984, 24)])

    plsc.subcore_barrier()

    c_off = jnp.full((16,), c * N, jnp.int32)
    ebase = c * EP

    def group(t8, carry):
        r8 = s * _CH + t8 * 8
        pltpu.sync_copy(src2.at[pl.ds(r8, 8)], src8)
        pltpu.sync_copy(dst2.at[pl.ds(r8, 8)], dst8)
        for jr in range(8):
            for j in range(_CK // 16):
                adj8[jr, pl.ds(j * 16, 16)] = (
                    src8[jr, pl.ds(j * 16, 16)] + c_off)

        def issue(jr, p):
            e = pltpu.async_copy(
                ea2f.at[pl.ds(ebase + (r8 + jr) * _CK, _CK)], mb[p], se[p])
            g = pltpu.async_copy(h2f.at[adj8.at[jr]], hb[p], sg[p])
            return e, g

        loads = {0: issue(0, 0)}
        scat = {}
        for jr in range(8):
            p = jr % 2
            e, g = loads.pop(jr)
            if jr < 7:
                loads[jr + 1] = issue(jr + 1, 1 - p)
            e.wait()
            g.wait()
            if jr >= 1:
                scat.pop(jr - 1).wait()

            def relu_row(i, icarry):
                for q in range(4):
                    ve = mb[p][i, pl.ds(q * 16, 16)]
                    vh = hb[p][i, pl.ds(q * 16, 16)]
                    elo = lax.bitcast_convert_type(ve << 16, jnp.float32)
                    ehi = lax.bitcast_convert_type(
                        ve & jnp.int32(-65536), jnp.float32)
                    hlo = lax.bitcast_convert_type(vh << 16, jnp.float32)
                    hhi = lax.bitcast_convert_type(
                        vh & jnp.int32(-65536), jnp.float32)
                    tbuf[i, pl.ds(q * 16, 16)] = jnp.maximum(hlo + elo, 0.0)
                    tbuf[i, pl.ds(64 + q * 16, 16)] = jnp.maximum(
                        hhi + ehi, 0.0)
                return icarry
            lax.fori_loop(0, _CK, relu_row, 0)
            scat[jr] = pltpu.async_copy(
                tbuf, aggr_sh.at[dst8.at[jr]], ssc, add=True)
        scat.pop(7).wait()
        return carry

    lax.fori_loop(0, _CH // 8, group, 0)
    plsc.subcore_barrier()

    pltpu.sync_copy(aggr_sh.at[pl.ds(s * 624, 624)],
                    out.at[pl.ds(c * N + s * 624, 624)])

    @pl.when(s == 0)
    def _():
        pltpu.sync_copy(aggr_sh.at[pl.ds(9984, 16)],
                        out.at[pl.ds(c * N + 9984, 16)])


def _sc_aggregate(h2f, ea2f, src2, dst2):
    mesh = plsc.VectorSubcoreMesh(core_axis_name="c", subcore_axis_name="s")
    kern = pl.kernel(
        _sc_aggr_body,
        out_type=jax.ShapeDtypeStruct((2 * N, 128), jnp.float32),
        mesh=mesh,
        compiler_params=pltpu.CompilerParams(use_tc_tiling_on_sc=False),
        scratch_types=[
            pltpu.VMEM((8, _CK), jnp.int32),
            pltpu.VMEM((8, _CK), jnp.int32),
            pltpu.VMEM((8, _CK), jnp.int32),
            pltpu.VMEM((_CK, 64), jnp.int32),
            pltpu.VMEM((_CK, 64), jnp.int32),
            pltpu.VMEM((_CK, 64), jnp.int32),
            pltpu.VMEM((_CK, 64), jnp.int32),
            pltpu.VMEM((_CK, 128), jnp.float32),
            pltpu.VMEM_SHARED((_SC_ROWS, 128), jnp.float32),
            pltpu.SemaphoreType.DMA,
            pltpu.SemaphoreType.DMA,
            pltpu.SemaphoreType.DMA,
            pltpu.SemaphoreType.DMA,
            pltpu.SemaphoreType.DMA,
        ],
    )
    return kern(h2f, ea2f, src2, dst2)


_BN = 1000
_NG = N // _BN
_BE = 2048
_EG = EP // _BE


def _embed_body(x_ref, w_ref, b_ref, out_ref):
    y = jnp.dot(x_ref[...], w_ref[...]) + b_ref[...]
    out_ref[0] = _pack_bf16_pairs(y[:, :128])
    out_ref[1] = _pack_bf16_pairs(y[:, 128:])


def _embed(x, W0, b0):
    return pl.pallas_call(
        _embed_body,
        grid=(_NG,),
        in_specs=[
            pl.BlockSpec((_BN, D), lambda i: (i, 0)),
            pl.BlockSpec((D, H), lambda i: (0, 0)),
            pl.BlockSpec((1, H), lambda i: (0, 0)),
        ],
        out_specs=pl.BlockSpec((2, _BN, 64), lambda i: (0, i, 0)),
        out_shape=jax.ShapeDtypeStruct((2, N, 64), jnp.int32),
    )(x, W0, b0.reshape(1, H))


def _pack_bf16_pairs(hh):
    rb = hh.astype(jnp.bfloat16).astype(jnp.float32)
    bi = lax.bitcast_convert_type(rb, jnp.int32)
    lo = lax.shift_right_logical(bi[:, :64], 16)
    hi = bi[:, 64:] & jnp.int32(-65536)
    return lo | hi


def _unpack_bf16_pairs(bi):
    lo = lax.bitcast_convert_type(bi << 16, jnp.float32)
    hi = lax.bitcast_convert_type(bi & jnp.int32(-65536), jnp.float32)
    return jnp.concatenate([lo, hi], axis=1)


def _ea_body(a_ref, w_ref, b_ref, out_ref):
    y = jnp.dot(a_ref[...], w_ref[...]) + b_ref[...]
    out_ref[0] = _pack_bf16_pairs(y[:, :128])
    out_ref[1] = _pack_bf16_pairs(y[:, 128:])


def _edge_proj(edge_attr_p, We_i, be_i):
    return pl.pallas_call(
        _ea_body,
        grid=(_EG,),
        in_specs=[
            pl.BlockSpec((_BE, ED), lambda i: (i, 0)),
            pl.BlockSpec((ED, H), lambda i: (0, 0)),
            pl.BlockSpec((1, H), lambda i: (0, 0)),
        ],
        out_specs=pl.BlockSpec((2, _BE, 64), lambda i: (0, i, 0)),
        out_shape=jax.ShapeDtypeStruct((2, EP, 64), jnp.int32),
    )(edge_attr_p, We_i, be_i.reshape(1, H))


def _mlp_body(h_ref, a_ref, w1_ref, b1_ref, w2_ref, b2_ref, si_ref,
              o_ref, stats_ref, acc):
    i = pl.program_id(0)
    h = jnp.concatenate(
        [_unpack_bf16_pairs(h_ref[0]), _unpack_bf16_pairs(h_ref[1])], axis=-1)
    a = jnp.concatenate([a_ref[0], a_ref[1]], axis=-1)
    an = a / jnp.maximum(
        jnp.sqrt(jnp.sum(a * a, axis=-1, keepdims=True)), 1e-12)
    hn = jnp.sqrt(jnp.sum(h * h, axis=-1, keepdims=True))
    out = h + an * (hn * si_ref[0, 0])
    t = jnp.maximum(jnp.dot(out, w1_ref[...]) + b1_ref[...], 0.0)
    o = jnp.dot(t, w2_ref[...]) + b2_ref[...]
    o_ref[...] = o
    blk = jnp.concatenate(
        [jnp.sum(o, axis=0, keepdims=True),
         jnp.sum(o * o, axis=0, keepdims=True),
         jnp.zeros((6, H), jnp.float32)], axis=0)

    @pl.when(i == 0)
    def _():
        acc[...] = blk

    @pl.when(i > 0)
    def _():
        acc[...] += blk

    @pl.when(i == _NG - 1)
    def _():
        stats_ref[...] = acc[...]


def _msgnorm_mlp(h2, aggr2, Wm1_i, bm1_i, Wm2_i, bm2_i, si):
    return pl.pallas_call(
        _mlp_body,
        grid=(_NG,),
        in_specs=[
            pl.BlockSpec((2, _BN, 64), lambda i: (0, i, 0)),
            pl.BlockSpec((2, _BN, 128), lambda i: (0, i, 0)),
            pl.BlockSpec((H, 2 * H), lambda i: (0, 0)),
            pl.BlockSpec((1, 2 * H), lambda i: (0, 0)),
            pl.BlockSpec((2 * H, H), lambda i: (0, 0)),
            pl.BlockSpec((1, H), lambda i: (0, 0)),
            pl.BlockSpec(memory_space=pltpu.SMEM),
        ],
        out_specs=[
            pl.BlockSpec((_BN, H), lambda i: (i, 0)),
            pl.BlockSpec((8, H), lambda i: (0, 0)),
        ],
        out_shape=[
            jax.ShapeDtypeStruct((N, H), jnp.float32),
            jax.ShapeDtypeStruct((8, H), jnp.float32),
        ],
        scratch_shapes=[pltpu.VMEM((8, H), jnp.float32)],
    )(h2, aggr2, Wm1_i, bm1_i.reshape(1, 2 * H), Wm2_i,
      bm2_i.reshape(1, H), si.reshape(1, 1))


def _bn_body(o_ref, st_ref, g_ref, b_ref, out_ref):
    mean = st_ref[0:1, :] * (1.0 / N)
    var = st_ref[1:2, :] * (1.0 / N) - mean * mean
    inv = lax.rsqrt(var + 1e-5)
    y = (o_ref[...] - mean) * (inv * g_ref[...]) + b_ref[...]
    hh = jnp.where(y > 0, y, 0.01 * y)
    out_ref[0] = _pack_bf16_pairs(hh[:, :128])
    out_ref[1] = _pack_bf16_pairs(hh[:, 128:])


def _batchnorm_leaky(o, stats, gamma_i, beta_i):
    return pl.pallas_call(
        _bn_body,
        grid=(_NG,),
        in_specs=[
            pl.BlockSpec((_BN, H), lambda i: (i, 0)),
            pl.BlockSpec((8, H), lambda i: (0, 0)),
            pl.BlockSpec((1, H), lambda i: (0, 0)),
            pl.BlockSpec((1, H), lambda i: (0, 0)),
        ],
        out_specs=pl.BlockSpec((2, _BN, 64), lambda i: (0, i, 0)),
        out_shape=jax.ShapeDtypeStruct((2, N, 64), jnp.int32),
    )(o, stats, gamma_i.reshape(1, H), beta_i.reshape(1, H))


def _pool_body(h_ref, b_ref, w1_ref, b1_ref, w2_ref, b2_ref, out_ref, acc):
    i = pl.program_id(0)
    h = jnp.concatenate(
        [_unpack_bf16_pairs(h_ref[0]), _unpack_bf16_pairs(h_ref[1])], axis=-1)
    brow = b_ref[0]
    gid = lax.broadcasted_iota(jnp.int32, (G, _BN), 0)
    oh = (gid == brow).astype(jnp.float32)
    part = jnp.dot(oh, h)

    @pl.when(i == 0)
    def _():
        acc[...] = part

    @pl.when(i > 0)
    def _():
        acc[...] += part

    r1 = jnp.dot(acc[...], w1_ref[...]) + b1_ref[...]
    r1 = jnp.where(r1 > 0, r1, 0.01 * r1)
    out_ref[...] = jnp.dot(r1, w2_ref[...]) + b2_ref[...]


def _pool_readout(h2, batch3, Wr1, br1, Wr2, br2):
    return pl.pallas_call(
        _pool_body,
        grid=(_NG,),
        in_specs=[
            pl.BlockSpec((2, _BN, 64), lambda i: (0, i, 0)),
            pl.BlockSpec((1, 1, _BN), lambda i: (i, 0, 0)),
            pl.BlockSpec((H, H // 2), lambda i: (0, 0)),
            pl.BlockSpec((1, H // 2), lambda i: (0, 0)),
            pl.BlockSpec((H // 2, 1), lambda i: (0, 0)),
            pl.BlockSpec((1, 1), lambda i: (0, 0)),
        ],
        out_specs=pl.BlockSpec((G, 1), lambda i: (0, 0)),
        out_shape=jax.ShapeDtypeStruct((G, 1), jnp.float32),
        scratch_shapes=[pltpu.VMEM((G, H), jnp.float32)],
    )(h2, batch3, Wr1, br1.reshape(1, H // 2), Wr2, br2.reshape(1, 1))


def kernel(x, edge_attr, W0, b0, We, be, Wm1, bm1, Wm2, bm2, s, gamma, beta,
           Wr1, br1, Wr2, br2, edge_index, batch):
    src = edge_index[0]
    dst = edge_index[1]
    pad = EP - E
    src2 = jnp.concatenate(
        [src, jnp.zeros((pad,), jnp.int32)]).reshape(EP // _CK, _CK)
    dst2 = jnp.concatenate(
        [dst, jnp.full((pad,), _TRASH, jnp.int32)]).reshape(EP // _CK, _CK)
    ea_p = jnp.concatenate([edge_attr, jnp.zeros((pad, ED), jnp.float32)])
    batch3 = batch.reshape(_NG, 1, _BN)

    h2 = _embed(x, W0, b0)
    for i in range(L):
        ea2 = _edge_proj(ea_p, We[i], be[i])
        aggr_f = _sc_aggregate(h2.reshape(2 * N, 64),
                               ea2.reshape(2 * EP, 64), src2, dst2)
        aggr2 = aggr_f.reshape(2, N, 128)
        o, stats = _msgnorm_mlp(h2, aggr2, Wm1[i], bm1[i], Wm2[i], bm2[i],
                                s[i])
        h2 = _batchnorm_leaky(o, stats, gamma[i], beta[i])
    return _pool_readout(h2, batch3, Wr1, br1, Wr2, br2)


# speedup vs baseline: 1.1516x; 1.1516x over previous
import functools

import jax
import jax.numpy as jnp
import numpy as np
from jax import lax
from jax.experimental import pallas as pl
from jax.experimental.pallas import tpu as pltpu
from jax.experimental.pallas import tpu_sc as plsc

N = 10000
E = 160000
EP = 163840
D = 256
ED = 16
H = 256
G = 64
L = 3

_CK = 64
_CH = 160
_ROWS_PER_TILE = N // 16
_TRASH = N
_SC_ROWS = N + 8


def _sc_aggr_body(h2f, ea2f, src2, dst2, out,
                  adj8, dst8, h0, h1, h2b, m0, m1,
                  aggr_sh, se0, se1, sg0, sg1, sg2, ssc):
    c = lax.axis_index("c")
    s = lax.axis_index("s")
    hb = (h0, h1, h2b)
    mb = (m0, m1)
    se = (se0, se1)
    sg = (sg0, sg1, sg2)

    def zrow(i, carry):
        for j in range(8):
            h0[i, pl.ds(j * 16, 16)] = jnp.zeros((16,), jnp.float32)
        return carry
    lax.fori_loop(0, _CK, zrow, 0)
    zbase = s * 624
    for k in range(10):
        off = k * 64
        nr = 64 if k < 9 else 48
        pltpu.sync_copy(h0.at[pl.ds(0, nr)],
                        aggr_sh.at[pl.ds(zbase + off, nr)])

    @pl.when(s == 0)
    def _():
        pltpu.sync_copy(h0.at[pl.ds(0, 24)], aggr_sh.at[pl.ds(9984, 24)])

    plsc.subcore_barrier()

    c_off = jnp.full((16,), c * N, jnp.int32)
    ebase = c * EP

    def group(t16, carry):
        r16 = s * _CH + t16 * 16
        pltpu.sync_copy(src2.at[pl.ds(r16, 16)], adj8)
        pltpu.sync_copy(dst2.at[pl.ds(r16, 16)], dst8)
        for jr in range(16):
            for j in range(_CK // 16):
                adj8[jr, pl.ds(j * 16, 16)] = (
                    adj8[jr, pl.ds(j * 16, 16)] + c_off)

        def issue_g(jr):
            p = jr % 3
            return pltpu.async_copy(h2f.at[adj8.at[jr]], hb[p], sg[p])

        def issue_e(jr):
            p = jr % 2
            return pltpu.async_copy(
                ea2f.at[pl.ds(ebase + (r16 + jr) * _CK, _CK)], mb[p], se[p])

        gl = {0: issue_g(0), 1: issue_g(1)}
        el = {0: issue_e(0), 1: issue_e(1)}
        scat = {}
        for jr in range(16):
            ph = jr % 3
            pe = jr % 2
            gl.pop(jr).wait()
            el.pop(jr).wait()
            if jr >= 1:
                scat.pop(jr - 1).wait()
            if jr < 14:
                gl[jr + 2] = issue_g(jr + 2)
            if jr < 15 and jr >= 1:
                el[jr + 1] = issue_e(jr + 1)

            def relu_row(i, icarry):
                for q in range(4):
                    ve = mb[pe][i, pl.ds(q * 16, 16)]
                    elo = lax.bitcast_convert_type(ve << 16, jnp.float32)
                    ehi = lax.bitcast_convert_type(
                        ve & jnp.int32(-65536), jnp.float32)
                    a0 = hb[ph][i, pl.ds(q * 16, 16)] + elo
                    a1 = hb[ph][i, pl.ds(64 + q * 16, 16)] + ehi
                    hb[ph][i, pl.ds(q * 16, 16)] = jnp.maximum(a0, 0.0)
                    hb[ph][i, pl.ds(64 + q * 16, 16)] = jnp.maximum(a1, 0.0)
                return icarry
            lax.fori_loop(0, _CK, relu_row, 0)
            scat[jr] = pltpu.async_copy(
                hb[ph], aggr_sh.at[dst8.at[jr]], ssc, add=True)
        scat.pop(15).wait()
        return carry

    lax.fori_loop(0, _CH // 16, group, 0)
    plsc.subcore_barrier()

    pltpu.sync_copy(aggr_sh.at[pl.ds(s * 624, 624)],
                    out.at[pl.ds(c * N + s * 624, 624)])

    @pl.when(s == 0)
    def _():
        pltpu.sync_copy(aggr_sh.at[pl.ds(9984, 16)],
                        out.at[pl.ds(c * N + 9984, 16)])


def _sc_aggregate(h2f, ea2f, src2, dst2):
    mesh = plsc.VectorSubcoreMesh(core_axis_name="c", subcore_axis_name="s")
    kern = pl.kernel(
        _sc_aggr_body,
        out_type=jax.ShapeDtypeStruct((2 * N, 128), jnp.float32),
        mesh=mesh,
        scratch_types=[
            pltpu.VMEM((16, _CK), jnp.int32),
            pltpu.VMEM((16, _CK), jnp.int32),
            pltpu.VMEM((_CK, 128), jnp.float32),
            pltpu.VMEM((_CK, 128), jnp.float32),
            pltpu.VMEM((_CK, 128), jnp.float32),
            pltpu.VMEM((_CK, 64), jnp.int32),
            pltpu.VMEM((_CK, 64), jnp.int32),
            pltpu.VMEM_SHARED((_SC_ROWS, 128), jnp.float32),
            pltpu.SemaphoreType.DMA,
            pltpu.SemaphoreType.DMA,
            pltpu.SemaphoreType.DMA,
            pltpu.SemaphoreType.DMA,
            pltpu.SemaphoreType.DMA,
            pltpu.SemaphoreType.DMA,
        ],
    )
    return kern(h2f, ea2f, src2, dst2)


_BN = 1000
_NG = N // _BN
_BE = 2048
_EG = EP // _BE


def _embed_body(x_ref, w_ref, b_ref, out_ref):
    y = jnp.dot(x_ref[...], w_ref[...]) + b_ref[...]
    out_ref[0] = y[:, :128]
    out_ref[1] = y[:, 128:]


def _embed(x, W0, b0):
    return pl.pallas_call(
        _embed_body,
        grid=(_NG,),
        in_specs=[
            pl.BlockSpec((_BN, D), lambda i: (i, 0)),
            pl.BlockSpec((D, H), lambda i: (0, 0)),
            pl.BlockSpec((1, H), lambda i: (0, 0)),
        ],
        out_specs=pl.BlockSpec((2, _BN, 128), lambda i: (0, i, 0)),
        out_shape=jax.ShapeDtypeStruct((2, N, 128), jnp.float32),
    )(x, W0, b0.reshape(1, H))


def _pack_bf16_pairs(hh):
    rb = hh.astype(jnp.bfloat16).astype(jnp.float32)
    bi = lax.bitcast_convert_type(rb, jnp.int32)
    lo = lax.shift_right_logical(bi[:, :64], 16)
    hi = bi[:, 64:] & jnp.int32(-65536)
    return lo | hi


def _unpack_bf16_pairs(bi):
    lo = lax.bitcast_convert_type(bi << 16, jnp.float32)
    hi = lax.bitcast_convert_type(bi & jnp.int32(-65536), jnp.float32)
    return jnp.concatenate([lo, hi], axis=1)


def _ea_body(a_ref, w_ref, b_ref, out_ref):
    y = jnp.dot(a_ref[...], w_ref[...]) + b_ref[...]
    out_ref[0] = _pack_bf16_pairs(y[:, :128])
    out_ref[1] = _pack_bf16_pairs(y[:, 128:])


def _edge_proj(edge_attr_p, We_i, be_i):
    return pl.pallas_call(
        _ea_body,
        grid=(_EG,),
        in_specs=[
            pl.BlockSpec((_BE, ED), lambda i: (i, 0)),
            pl.BlockSpec((ED, H), lambda i: (0, 0)),
            pl.BlockSpec((1, H), lambda i: (0, 0)),
        ],
        out_specs=pl.BlockSpec((2, _BE, 64), lambda i: (0, i, 0)),
        out_shape=jax.ShapeDtypeStruct((2, EP, 64), jnp.int32),
    )(edge_attr_p, We_i, be_i.reshape(1, H))


def _mlp_body(h_ref, a_ref, w1_ref, b1_ref, w2_ref, b2_ref, si_ref,
              o_ref, stats_ref, acc):
    i = pl.program_id(0)
    h = jnp.concatenate([h_ref[0], h_ref[1]], axis=-1)
    a = jnp.concatenate([a_ref[0], a_ref[1]], axis=-1)
    an = a / jnp.maximum(
        jnp.sqrt(jnp.sum(a * a, axis=-1, keepdims=True)), 1e-12)
    hn = jnp.sqrt(jnp.sum(h * h, axis=-1, keepdims=True))
    out = h + an * (hn * si_ref[0, 0])
    t = jnp.maximum(jnp.dot(out, w1_ref[...]) + b1_ref[...], 0.0)
    o = jnp.dot(t, w2_ref[...]) + b2_ref[...]
    o_ref[...] = o
    blk = jnp.concatenate(
        [jnp.sum(o, axis=0, keepdims=True),
         jnp.sum(o * o, axis=0, keepdims=True),
         jnp.zeros((6, H), jnp.float32)], axis=0)

    @pl.when(i == 0)
    def _():
        acc[...] = blk

    @pl.when(i > 0)
    def _():
        acc[...] += blk

    @pl.when(i == _NG - 1)
    def _():
        stats_ref[...] = acc[...]


def _msgnorm_mlp(h2, aggr2, Wm1_i, bm1_i, Wm2_i, bm2_i, si):
    return pl.pallas_call(
        _mlp_body,
        grid=(_NG,),
        in_specs=[
            pl.BlockSpec((2, _BN, 128), lambda i: (0, i, 0)),
            pl.BlockSpec((2, _BN, 128), lambda i: (0, i, 0)),
            pl.BlockSpec((H, 2 * H), lambda i: (0, 0)),
            pl.BlockSpec((1, 2 * H), lambda i: (0, 0)),
            pl.BlockSpec((2 * H, H), lambda i: (0, 0)),
            pl.BlockSpec((1, H), lambda i: (0, 0)),
            pl.BlockSpec(memory_space=pltpu.SMEM),
        ],
        out_specs=[
            pl.BlockSpec((_BN, H), lambda i: (i, 0)),
            pl.BlockSpec((8, H), lambda i: (0, 0)),
        ],
        out_shape=[
            jax.ShapeDtypeStruct((N, H), jnp.float32),
            jax.ShapeDtypeStruct((8, H), jnp.float32),
        ],
        scratch_shapes=[pltpu.VMEM((8, H), jnp.float32)],
    )(h2, aggr2, Wm1_i, bm1_i.reshape(1, 2 * H), Wm2_i,
      bm2_i.reshape(1, H), si.reshape(1, 1))


def _bn_body(o_ref, st_ref, g_ref, b_ref, out_ref):
    mean = st_ref[0:1, :] * (1.0 / N)
    var = st_ref[1:2, :] * (1.0 / N) - mean * mean
    inv = lax.rsqrt(var + 1e-5)
    y = (o_ref[...] - mean) * (inv * g_ref[...]) + b_ref[...]
    hh = jnp.where(y > 0, y, 0.01 * y)
    out_ref[0] = hh[:, :128]
    out_ref[1] = hh[:, 128:]


def _batchnorm_leaky(o, stats, gamma_i, beta_i):
    return pl.pallas_call(
        _bn_body,
        grid=(_NG,),
        in_specs=[
            pl.BlockSpec((_BN, H), lambda i: (i, 0)),
            pl.BlockSpec((8, H), lambda i: (0, 0)),
            pl.BlockSpec((1, H), lambda i: (0, 0)),
            pl.BlockSpec((1, H), lambda i: (0, 0)),
        ],
        out_specs=pl.BlockSpec((2, _BN, 128), lambda i: (0, i, 0)),
        out_shape=jax.ShapeDtypeStruct((2, N, 128), jnp.float32),
    )(o, stats, gamma_i.reshape(1, H), beta_i.reshape(1, H))


def _pool_body(h_ref, b_ref, w1_ref, b1_ref, w2_ref, b2_ref, out_ref, acc):
    i = pl.program_id(0)
    h = jnp.concatenate([h_ref[0], h_ref[1]], axis=-1)
    brow = b_ref[0]
    gid = lax.broadcasted_iota(jnp.int32, (G, _BN), 0)
    oh = (gid == brow).astype(jnp.float32)
    part = jnp.dot(oh, h)

    @pl.when(i == 0)
    def _():
        acc[...] = part

    @pl.when(i > 0)
    def _():
        acc[...] += part

    r1 = jnp.dot(acc[...], w1_ref[...]) + b1_ref[...]
    r1 = jnp.where(r1 > 0, r1, 0.01 * r1)
    out_ref[...] = jnp.dot(r1, w2_ref[...]) + b2_ref[...]


def _pool_readout(h2, batch3, Wr1, br1, Wr2, br2):
    return pl.pallas_call(
        _pool_body,
        grid=(_NG,),
        in_specs=[
            pl.BlockSpec((2, _BN, 128), lambda i: (0, i, 0)),
            pl.BlockSpec((1, 1, _BN), lambda i: (i, 0, 0)),
            pl.BlockSpec((H, H // 2), lambda i: (0, 0)),
            pl.BlockSpec((1, H // 2), lambda i: (0, 0)),
            pl.BlockSpec((H // 2, 1), lambda i: (0, 0)),
            pl.BlockSpec((1, 1), lambda i: (0, 0)),
        ],
        out_specs=pl.BlockSpec((G, 1), lambda i: (0, 0)),
        out_shape=jax.ShapeDtypeStruct((G, 1), jnp.float32),
        scratch_shapes=[pltpu.VMEM((G, H), jnp.float32)],
    )(h2, batch3, Wr1, br1.reshape(1, H // 2), Wr2, br2.reshape(1, 1))


def kernel(x, edge_attr, W0, b0, We, be, Wm1, bm1, Wm2, bm2, s, gamma, beta,
           Wr1, br1, Wr2, br2, edge_index, batch):
    src = edge_index[0]
    dst = edge_index[1]
    pad = EP - E
    src2 = jnp.concatenate(
        [src, jnp.zeros((pad,), jnp.int32)]).reshape(EP // _CK, _CK)
    dst2 = jnp.concatenate(
        [dst, jnp.full((pad,), _TRASH, jnp.int32)]).reshape(EP // _CK, _CK)
    ea_p = jnp.concatenate([edge_attr, jnp.zeros((pad, ED), jnp.float32)])
    batch3 = batch.reshape(_NG, 1, _BN)

    h2 = _embed(x, W0, b0)
    for i in range(L):
        ea2 = _edge_proj(ea_p, We[i], be[i])
        aggr_f = _sc_aggregate(h2.reshape(2 * N, 128),
                               ea2.reshape(2 * EP, 64), src2, dst2)
        aggr2 = aggr_f.reshape(2, N, 128)
        o, stats = _msgnorm_mlp(h2, aggr2, Wm1[i], bm1[i], Wm2[i], bm2[i],
                                s[i])
        h2 = _batchnorm_leaky(o, stats, gamma[i], beta[i])
    return _pool_readout(h2, batch3, Wr1, br1, Wr2, br2)
